# baseline (speedup 1.0000x reference)
import functools

import jax
import jax.numpy as jnp
from jax import lax
from jax.experimental import pallas as pl
from jax.experimental.pallas import tpu as pltpu
from jax.experimental.pallas import tpu_sc as plsc

B_ROWS = 16384
B_COLS = 26
DIM = 64
B_FLAT = B_ROWS * B_COLS

_info = plsc.get_sparse_core_info()
NC = _info.num_cores
NS = _info.num_subcores
NW = NC * NS
B_PER_W = B_FLAT // NW
CHUNK = 1024
N_CHUNKS = B_PER_W // CHUNK

_mesh = plsc.VectorSubcoreMesh(core_axis_name="c", subcore_axis_name="s")


@functools.partial(
    pl.kernel,
    out_type=jax.ShapeDtypeStruct((B_FLAT, DIM), jnp.float32),
    mesh=_mesh,
    scratch_types=[
        pltpu.VMEM((CHUNK,), jnp.int32),
        pltpu.VMEM((CHUNK, DIM), jnp.float32),
        pltpu.SemaphoreType.DMA,
    ],
    compiler_params=pltpu.CompilerParams(use_tc_tiling_on_sc=False),
)
def _gather_kernel(x_hbm, tab_hbm, out_hbm, idx_v, rows_v, sem):
    wid = lax.axis_index("s") * NC + lax.axis_index("c")
    base = wid * B_PER_W

    def step(i, carry):
        off = base + i * CHUNK
        pltpu.sync_copy(x_hbm.at[pl.ds(off, CHUNK)], idx_v)
        pltpu.async_copy(tab_hbm.at[idx_v], rows_v, sem).wait()
        pltpu.sync_copy(rows_v, out_hbm.at[pl.ds(off, CHUNK)])
        return carry

    lax.fori_loop(0, N_CHUNKS, step, 0)


def kernel(x, emb_weight):
    x_flat = x.reshape(B_FLAT).astype(jnp.int32)
    out = _gather_kernel(x_flat, emb_weight)
    return out.reshape(B_ROWS, B_COLS, DIM)


# speedup vs baseline: 1.0113x; 1.0113x over previous
import functools

import jax
import jax.numpy as jnp
from jax import lax
from jax.experimental import pallas as pl
from jax.experimental.pallas import tpu as pltpu
from jax.experimental.pallas import tpu_sc as plsc

B_ROWS = 16384
B_COLS = 26
DIM = 64
B_FLAT = B_ROWS * B_COLS

_info = plsc.get_sparse_core_info()
NC = _info.num_cores
NS = _info.num_subcores
NW = NC * NS
B_PER_W = B_FLAT // NW
CHUNK = 512
N_CHUNKS = B_PER_W // CHUNK
NBUF = 2

_mesh = plsc.VectorSubcoreMesh(core_axis_name="c", subcore_axis_name="s")


@functools.partial(
    pl.kernel,
    out_type=jax.ShapeDtypeStruct((B_FLAT, DIM), jnp.float32),
    mesh=_mesh,
    scratch_types=[
        pltpu.VMEM((CHUNK,), jnp.int32),
        pltpu.VMEM((CHUNK,), jnp.int32),
        pltpu.VMEM((CHUNK, DIM), jnp.float32),
        pltpu.VMEM((CHUNK, DIM), jnp.float32),
        pltpu.SemaphoreType.DMA,
        pltpu.SemaphoreType.DMA,
        pltpu.SemaphoreType.DMA,
        pltpu.SemaphoreType.DMA,
    ],
    compiler_params=pltpu.CompilerParams(use_tc_tiling_on_sc=False),
)
def _gather_kernel(x_hbm, tab_hbm, out_hbm, idx0, idx1, rows0, rows1,
                   gsem0, gsem1, osem0, osem1):
    wid = lax.axis_index("s") * NC + lax.axis_index("c")
    base = wid * B_PER_W
    bufs = ((idx0, rows0, gsem0, osem0), (idx1, rows1, gsem1, osem1))

    for b in range(NBUF):
        idx_v, rows_v, gsem, _ = bufs[b]
        off = base + b * CHUNK
        pltpu.sync_copy(x_hbm.at[pl.ds(off, CHUNK)], idx_v)
        pltpu.async_copy(tab_hbm.at[idx_v], rows_v, gsem)

    def outer(j, carry):
        for b in range(NBUF):
            i = j * NBUF + b
            idx_v, rows_v, gsem, osem = bufs[b]
            off = base + i * CHUNK
            pltpu.make_async_copy(tab_hbm.at[idx_v], rows_v, gsem).wait()
            pltpu.async_copy(rows_v, out_hbm.at[pl.ds(off, CHUNK)], osem)

            @pl.when(i + NBUF < N_CHUNKS)
            def _():
                off2 = base + (i + NBUF) * CHUNK
                pltpu.sync_copy(x_hbm.at[pl.ds(off2, CHUNK)], idx_v)
                pltpu.make_async_copy(
                    rows_v, out_hbm.at[pl.ds(off, CHUNK)], osem).wait()
                pltpu.async_copy(tab_hbm.at[idx_v], rows_v, gsem)

            @pl.when(i + NBUF >= N_CHUNKS)
            def _():
                pltpu.make_async_copy(
                    rows_v, out_hbm.at[pl.ds(off, CHUNK)], osem).wait()

        return carry

    lax.fori_loop(0, N_CHUNKS // NBUF, outer, 0)


def kernel(x, emb_weight):
    x_flat = x.reshape(B_FLAT).astype(jnp.int32)
    out = _gather_kernel(x_flat, emb_weight)
    return out.reshape(B_ROWS, B_COLS, DIM)
